# baseline (speedup 1.0000x reference)
import numpy as np
import jax
import jax.numpy as jnp
from jax import lax
from jax.experimental import pallas as pl
from jax.experimental.pallas import tpu as pltpu

_FEATURE_STRIDE = 16
_NUM_ANCHORS = 9
_PRE_NMS_TOPN = 6000
_POST_NMS_TOPN = 300
_NMS_THRESH = 0.7
_MIN_SIZE = 16.0
_H = 64
_W = 64
_HW = _H * _W
_N = _HW * _NUM_ANCHORS
_ROWS = _N // 128

_NEG_INF = np.float32(-np.inf)


def _base_anchor_consts():
    base_size = 16
    ratios = (0.5, 1.0, 2.0)
    scales = (8.0, 16.0, 32.0)
    w = float(base_size)
    h = float(base_size)
    x_ctr = 0.5 * (w - 1.0)
    y_ctr = 0.5 * (h - 1.0)
    size = w * h
    anchors = []
    for r in ratios:
        ws = np.round(np.sqrt(size / r))
        hs = np.round(ws * r)
        for s in scales:
            ws2 = ws * s
            hs2 = hs * s
            anchors.append([x_ctr - 0.5 * (ws2 - 1.0), y_ctr - 0.5 * (hs2 - 1.0),
                            x_ctr + 0.5 * (ws2 - 1.0), y_ctr + 0.5 * (hs2 - 1.0)])
    a = np.array(anchors, dtype=np.float32)
    widths = a[:, 2] - a[:, 0] + np.float32(1.0)
    heights = a[:, 3] - a[:, 1] + np.float32(1.0)
    ctr_x = a[:, 0] + np.float32(0.5) * widths
    ctr_y = a[:, 1] + np.float32(0.5) * heights
    return (widths.reshape(9, 1), heights.reshape(9, 1),
            ctr_x.reshape(9, 1), ctr_y.reshape(9, 1))


_AW, _AH, _ACX, _ACY = _base_anchor_consts()


def _trunk_kernel(im_ref, fm_ref, w1_ref, b1_ref, wcb_ref, bcb_ref, ac_ref,
                  out_ref):
    acc = jnp.zeros((_HW, 256), jnp.float32)
    for i in range(9):
        ky, kx = i // 3, i % 3
        sl = fm_ref[ky:ky + _H, kx:kx + _W, :]
        a = sl.reshape(_HW, 256)
        acc = acc + jnp.dot(a, w1_ref[i], preferred_element_type=jnp.float32)
    x = jnp.maximum(acc + b1_ref[0:1, :], 0.0)
    out2 = jnp.dot(x, wcb_ref[...], preferred_element_type=jnp.float32)
    out2 = out2 + bcb_ref[0:1, :]
    t = out2.T

    l1 = t[0:9, :]
    l2 = t[9:18, :]
    dx = t[18:27, :]
    dy = t[27:36, :]
    dw = t[36:45, :]
    dh = t[45:54, :]

    m = jnp.maximum(l1, l2)
    e1 = jnp.exp(l1 - m)
    e2 = jnp.exp(l2 - m)
    score = e2 / (e1 + e2)

    aw = ac_ref[:, 0:1]
    ah = ac_ref[:, 1:2]
    acx = ac_ref[:, 2:3]
    acy = ac_ref[:, 3:4]

    hw = lax.broadcasted_iota(jnp.int32, (_NUM_ANCHORS, _HW), 1)
    sx = ((hw % _W) * _FEATURE_STRIDE).astype(jnp.float32)
    sy = ((hw // _W) * _FEATURE_STRIDE).astype(jnp.float32)
    ctr_x = sx + acx
    ctr_y = sy + acy

    dw = jnp.clip(dw, -10.0, 10.0)
    dh = jnp.clip(dh, -10.0, 10.0)
    pcx = dx * aw + ctr_x
    pcy = dy * ah + ctr_y
    pw = jnp.exp(dw) * aw
    ph = jnp.exp(dh) * ah

    im_h = im_ref[0, 0]
    im_w = im_ref[0, 1]
    im_scale = im_ref[0, 2]
    x1 = jnp.clip(pcx - 0.5 * pw, 0.0, im_w - 1.0)
    y1 = jnp.clip(pcy - 0.5 * ph, 0.0, im_h - 1.0)
    x2 = jnp.clip(pcx + 0.5 * pw, 0.0, im_w - 1.0)
    y2 = jnp.clip(pcy + 0.5 * ph, 0.0, im_h - 1.0)
    ws_ = x2 - x1 + 1.0
    hs_ = y2 - y1 + 1.0
    valid = (ws_ >= _MIN_SIZE * im_scale) & (hs_ >= _MIN_SIZE * im_scale)
    s_masked = jnp.where(valid, score, _NEG_INF)

    out_ref[0:9, :] = s_masked
    out_ref[9:18, :] = x1
    out_ref[18:27, :] = y1
    out_ref[27:36, :] = x2
    out_ref[36:45, :] = y2


def _nms_kernel(s_ref, x1_ref, y1_ref, x2_ref, y2_ref, bi_ref, out_ref):
    s_in = s_ref[...]
    si = lax.bitcast_convert_type(s_in, jnp.int32)

    lo0 = jnp.int32(np.int32(np.float32(-np.inf).view(np.int32)))
    hi0 = jnp.int32(np.float32(1.0).view(np.int32) + 1)

    def bs_body(_, carry):
        lo, hi = carry
        mid = (lo + hi) // 2
        cnt = jnp.sum(jnp.where(si >= mid, 1.0, 0.0))
        take = cnt >= float(_PRE_NMS_TOPN)
        lo = jnp.where(take, mid, lo)
        hi = jnp.where(take, hi, mid)
        return (lo, hi)

    lo, _ = lax.fori_loop(0, 31, bs_body, (lo0, hi0))
    s0 = jnp.where(si >= lo, s_in, _NEG_INF)

    lin = (lax.broadcasted_iota(jnp.int32, (_ROWS, 128), 0) * 128
           + lax.broadcasted_iota(jnp.int32, (_ROWS, 128), 1))
    li128 = lax.broadcasted_iota(jnp.int32, (1, 128), 1)
    li5 = lax.broadcasted_iota(jnp.int32, (1, 5), 1)
    big = jnp.int32(2 ** 30)

    bx1 = x1_ref[...]
    by1 = y1_ref[...]
    bx2 = x2_ref[...]
    by2 = y2_ref[...]
    areas = (bx2 - bx1 + 1.0) * (by2 - by1 + 1.0)

    def body(i, carry):
        s, idx0 = carry
        mx = jnp.max(s)
        idx = jnp.min(jnp.where(s == mx, lin, big))
        idx_eff = jnp.where(mx > _NEG_INF, idx,
                            jnp.where(i == 0, 0, idx0))
        idx0 = jnp.where(i == 0, idx_eff, idx0)
        r = idx_eff // 128
        c = idx_eff % 128
        rows = bi_ref[pl.ds(4 * r, 4), :]
        p = jnp.sum(jnp.where(li128 == c, rows, 0.0), axis=1, keepdims=True)
        px1 = p[0:1, 0:1]
        py1 = p[1:2, 0:1]
        px2 = p[2:3, 0:1]
        py2 = p[3:4, 0:1]

        pa = (px2 - px1 + 1.0) * (py2 - py1 + 1.0)
        xx1 = jnp.maximum(px1, bx1)
        yy1 = jnp.maximum(py1, by1)
        xx2 = jnp.minimum(px2, bx2)
        yy2 = jnp.minimum(py2, by2)
        iw = jnp.maximum(0.0, xx2 - xx1 + 1.0)
        ih = jnp.maximum(0.0, yy2 - yy1 + 1.0)
        inter = iw * ih
        iou = inter / (areas + pa - inter)
        s = jnp.where(iou > _NMS_THRESH, _NEG_INF, s)
        s = jnp.where(lin == idx_eff, _NEG_INF, s)

        row = jnp.where(li5 == 1, px1,
                        jnp.where(li5 == 2, py1,
                                  jnp.where(li5 == 3, px2,
                                            jnp.where(li5 == 4, py2, 0.0))))
        out_ref[pl.ds(i, 1), :] = row
        return (s, idx0)

    lax.fori_loop(0, _POST_NMS_TOPN, body, (s0, jnp.int32(0)))


def kernel(feature_map, im_info, W1, b1, Wc, bc, Wb, bb):
    fm = jnp.transpose(feature_map[0], (1, 2, 0))
    fm_p = jnp.pad(fm, ((1, 1), (1, 1), (0, 0)))
    w1r = jnp.transpose(W1, (2, 3, 1, 0)).reshape(9, 256, 256)
    wc_t = Wc[:, :, 0, 0].T
    wb_t = Wb[:, :, 0, 0].reshape(9, 4, 256).transpose(1, 0, 2).reshape(36, 256).T
    wcb = jnp.concatenate([wc_t, wb_t], axis=1)
    wcb = jnp.pad(wcb, ((0, 0), (0, 128 - 54)))
    bcb = jnp.concatenate([bc, bb.reshape(9, 4).T.reshape(36),
                           jnp.zeros((128 - 54,), jnp.float32)]).reshape(1, 128)
    b1r = b1.reshape(1, 256)
    ac = jnp.asarray(np.concatenate([_AW, _AH, _ACX, _ACY], axis=1))

    packed = pl.pallas_call(
        _trunk_kernel,
        out_shape=jax.ShapeDtypeStruct((45, _HW), jnp.float32),
        in_specs=[
            pl.BlockSpec(memory_space=pltpu.SMEM),
            pl.BlockSpec(memory_space=pltpu.VMEM),
            pl.BlockSpec(memory_space=pltpu.VMEM),
            pl.BlockSpec(memory_space=pltpu.VMEM),
            pl.BlockSpec(memory_space=pltpu.VMEM),
            pl.BlockSpec(memory_space=pltpu.VMEM),
            pl.BlockSpec(memory_space=pltpu.VMEM),
        ],
        out_specs=pl.BlockSpec(memory_space=pltpu.VMEM),
    )(im_info, fm_p, w1r, b1r, wcb, bcb, ac)

    s = packed[0:9].reshape(_ROWS, 128)
    x1 = packed[9:18].reshape(_ROWS, 128)
    y1 = packed[18:27].reshape(_ROWS, 128)
    x2 = packed[27:36].reshape(_ROWS, 128)
    y2 = packed[36:45].reshape(_ROWS, 128)

    bint = jnp.stack([x1, y1, x2, y2], axis=1).reshape(_ROWS * 4, 128)
    rois = pl.pallas_call(
        _nms_kernel,
        out_shape=jax.ShapeDtypeStruct((_POST_NMS_TOPN, 5), jnp.float32),
    )(s, x1, y1, x2, y2, bint)
    return rois


# speedup vs baseline: 1.0151x; 1.0151x over previous
import numpy as np
import jax
import jax.numpy as jnp
from jax import lax
from jax.experimental import pallas as pl
from jax.experimental.pallas import tpu as pltpu

_FEATURE_STRIDE = 16
_NUM_ANCHORS = 9
_PRE_NMS_TOPN = 6000
_POST_NMS_TOPN = 300
_NMS_THRESH = 0.7
_MIN_SIZE = 16.0
_H = 64
_W = 64
_HW = _H * _W
_N = _HW * _NUM_ANCHORS
_ROWS = _N // 128

_NEG_INF = np.float32(-np.inf)


def _base_anchor_consts():
    base_size = 16
    ratios = (0.5, 1.0, 2.0)
    scales = (8.0, 16.0, 32.0)
    w = float(base_size)
    h = float(base_size)
    x_ctr = 0.5 * (w - 1.0)
    y_ctr = 0.5 * (h - 1.0)
    size = w * h
    anchors = []
    for r in ratios:
        ws = np.round(np.sqrt(size / r))
        hs = np.round(ws * r)
        for s in scales:
            ws2 = ws * s
            hs2 = hs * s
            anchors.append([x_ctr - 0.5 * (ws2 - 1.0), y_ctr - 0.5 * (hs2 - 1.0),
                            x_ctr + 0.5 * (ws2 - 1.0), y_ctr + 0.5 * (hs2 - 1.0)])
    a = np.array(anchors, dtype=np.float32)
    widths = a[:, 2] - a[:, 0] + np.float32(1.0)
    heights = a[:, 3] - a[:, 1] + np.float32(1.0)
    ctr_x = a[:, 0] + np.float32(0.5) * widths
    ctr_y = a[:, 1] + np.float32(0.5) * heights
    return (widths.reshape(9, 1), heights.reshape(9, 1),
            ctr_x.reshape(9, 1), ctr_y.reshape(9, 1))


_AW, _AH, _ACX, _ACY = _base_anchor_consts()


def _trunk_kernel(im_ref, fm_ref, w1_ref, b1_ref, wcb_ref, bcb_ref, ac_ref,
                  out_ref):
    acc = jnp.zeros((_HW, 256), jnp.float32)
    for i in range(9):
        ky, kx = i // 3, i % 3
        sl = fm_ref[ky:ky + _H, kx:kx + _W, :]
        a = sl.reshape(_HW, 256)
        acc = acc + jnp.dot(a, w1_ref[i], preferred_element_type=jnp.float32)
    x = jnp.maximum(acc + b1_ref[0:1, :], 0.0)
    out2 = jnp.dot(x, wcb_ref[...], preferred_element_type=jnp.float32)
    out2 = out2 + bcb_ref[0:1, :]
    t = out2.T

    l1 = t[0:9, :]
    l2 = t[9:18, :]
    dx = t[18:27, :]
    dy = t[27:36, :]
    dw = t[36:45, :]
    dh = t[45:54, :]

    m = jnp.maximum(l1, l2)
    e1 = jnp.exp(l1 - m)
    e2 = jnp.exp(l2 - m)
    score = e2 / (e1 + e2)

    aw = ac_ref[:, 0:1]
    ah = ac_ref[:, 1:2]
    acx = ac_ref[:, 2:3]
    acy = ac_ref[:, 3:4]

    hw = lax.broadcasted_iota(jnp.int32, (_NUM_ANCHORS, _HW), 1)
    sx = ((hw % _W) * _FEATURE_STRIDE).astype(jnp.float32)
    sy = ((hw // _W) * _FEATURE_STRIDE).astype(jnp.float32)
    ctr_x = sx + acx
    ctr_y = sy + acy

    dw = jnp.clip(dw, -10.0, 10.0)
    dh = jnp.clip(dh, -10.0, 10.0)
    pcx = dx * aw + ctr_x
    pcy = dy * ah + ctr_y
    pw = jnp.exp(dw) * aw
    ph = jnp.exp(dh) * ah

    im_h = im_ref[0, 0]
    im_w = im_ref[0, 1]
    im_scale = im_ref[0, 2]
    x1 = jnp.clip(pcx - 0.5 * pw, 0.0, im_w - 1.0)
    y1 = jnp.clip(pcy - 0.5 * ph, 0.0, im_h - 1.0)
    x2 = jnp.clip(pcx + 0.5 * pw, 0.0, im_w - 1.0)
    y2 = jnp.clip(pcy + 0.5 * ph, 0.0, im_h - 1.0)
    ws_ = x2 - x1 + 1.0
    hs_ = y2 - y1 + 1.0
    valid = (ws_ >= _MIN_SIZE * im_scale) & (hs_ >= _MIN_SIZE * im_scale)
    s_masked = jnp.where(valid, score, _NEG_INF)

    out_ref[0:9, :] = s_masked
    out_ref[9:18, :] = x1
    out_ref[18:27, :] = y1
    out_ref[27:36, :] = x2
    out_ref[36:45, :] = y2


def _nms_kernel(s_ref, x1_ref, y1_ref, x2_ref, y2_ref, out_ref):
    s_in = s_ref[...]
    si = lax.bitcast_convert_type(s_in, jnp.int32)

    lo0 = jnp.int32(np.int32(np.float32(-np.inf).view(np.int32)))
    hi0 = jnp.int32(np.float32(1.0).view(np.int32) + 1)

    def bs_body(_, carry):
        lo, hi = carry
        mid = (lo + hi) // 2
        cnt = jnp.sum(jnp.where(si >= mid, 1.0, 0.0))
        take = cnt >= float(_PRE_NMS_TOPN)
        lo = jnp.where(take, mid, lo)
        hi = jnp.where(take, hi, mid)
        return (lo, hi)

    lo, _ = lax.fori_loop(0, 31, bs_body, (lo0, hi0))
    s0 = jnp.where(si >= lo, s_in, _NEG_INF)

    lin = (lax.broadcasted_iota(jnp.int32, (_ROWS, 128), 0) * 128
           + lax.broadcasted_iota(jnp.int32, (_ROWS, 128), 1))
    li128 = lax.broadcasted_iota(jnp.int32, (1, 128), 1)
    li5 = lax.broadcasted_iota(jnp.int32, (1, 5), 1)
    big = jnp.int32(2 ** 30)

    bx1 = x1_ref[...]
    by1 = y1_ref[...]
    bx2 = x2_ref[...]
    by2 = y2_ref[...]
    areas = (bx2 - bx1 + 1.0) * (by2 - by1 + 1.0)

    def body(i, carry):
        s, idx0 = carry
        mx = jnp.max(s)
        idx = jnp.min(jnp.where(s == mx, lin, big))
        idx_eff = jnp.where(mx > _NEG_INF, idx,
                            jnp.where(i == 0, 0, idx0))
        idx0 = jnp.where(i == 0, idx_eff, idx0)
        r = idx_eff // 128
        c = idx_eff % 128
        rows = jnp.concatenate([x1_ref[pl.ds(r, 1), :], y1_ref[pl.ds(r, 1), :],
                                x2_ref[pl.ds(r, 1), :], y2_ref[pl.ds(r, 1), :]],
                               axis=0)
        p = jnp.sum(jnp.where(li128 == c, rows, 0.0), axis=1, keepdims=True)
        px1 = p[0:1, 0:1]
        py1 = p[1:2, 0:1]
        px2 = p[2:3, 0:1]
        py2 = p[3:4, 0:1]

        pa = (px2 - px1 + 1.0) * (py2 - py1 + 1.0)
        xx1 = jnp.maximum(px1, bx1)
        yy1 = jnp.maximum(py1, by1)
        xx2 = jnp.minimum(px2, bx2)
        yy2 = jnp.minimum(py2, by2)
        iw = jnp.maximum(0.0, xx2 - xx1 + 1.0)
        ih = jnp.maximum(0.0, yy2 - yy1 + 1.0)
        inter = iw * ih
        iou = inter / (areas + pa - inter)
        s = jnp.where(iou > _NMS_THRESH, _NEG_INF, s)
        s = jnp.where(lin == idx_eff, _NEG_INF, s)

        row = jnp.where(li5 == 1, px1,
                        jnp.where(li5 == 2, py1,
                                  jnp.where(li5 == 3, px2,
                                            jnp.where(li5 == 4, py2, 0.0))))
        out_ref[pl.ds(i, 1), :] = row
        return (s, idx0)

    lax.fori_loop(0, _POST_NMS_TOPN, body, (s0, jnp.int32(0)), unroll=2)


def kernel(feature_map, im_info, W1, b1, Wc, bc, Wb, bb):
    fm = jnp.transpose(feature_map[0], (1, 2, 0))
    fm_p = jnp.pad(fm, ((1, 1), (1, 1), (0, 0)))
    w1r = jnp.transpose(W1, (2, 3, 1, 0)).reshape(9, 256, 256)
    wc_t = Wc[:, :, 0, 0].T
    wb_t = Wb[:, :, 0, 0].reshape(9, 4, 256).transpose(1, 0, 2).reshape(36, 256).T
    wcb = jnp.concatenate([wc_t, wb_t], axis=1)
    wcb = jnp.pad(wcb, ((0, 0), (0, 128 - 54)))
    bcb = jnp.concatenate([bc, bb.reshape(9, 4).T.reshape(36),
                           jnp.zeros((128 - 54,), jnp.float32)]).reshape(1, 128)
    b1r = b1.reshape(1, 256)
    ac = jnp.asarray(np.concatenate([_AW, _AH, _ACX, _ACY], axis=1))

    packed = pl.pallas_call(
        _trunk_kernel,
        out_shape=jax.ShapeDtypeStruct((45, _HW), jnp.float32),
        in_specs=[
            pl.BlockSpec(memory_space=pltpu.SMEM),
            pl.BlockSpec(memory_space=pltpu.VMEM),
            pl.BlockSpec(memory_space=pltpu.VMEM),
            pl.BlockSpec(memory_space=pltpu.VMEM),
            pl.BlockSpec(memory_space=pltpu.VMEM),
            pl.BlockSpec(memory_space=pltpu.VMEM),
            pl.BlockSpec(memory_space=pltpu.VMEM),
        ],
        out_specs=pl.BlockSpec(memory_space=pltpu.VMEM),
    )(im_info, fm_p, w1r, b1r, wcb, bcb, ac)

    s = packed[0:9].reshape(_ROWS, 128)
    x1 = packed[9:18].reshape(_ROWS, 128)
    y1 = packed[18:27].reshape(_ROWS, 128)
    x2 = packed[27:36].reshape(_ROWS, 128)
    y2 = packed[36:45].reshape(_ROWS, 128)

    rois = pl.pallas_call(
        _nms_kernel,
        out_shape=jax.ShapeDtypeStruct((_POST_NMS_TOPN, 5), jnp.float32),
    )(s, x1, y1, x2, y2)
    return rois
